# baseline (speedup 1.0000x reference)
import functools

import jax
import jax.numpy as jnp
from jax import lax
from jax.experimental import pallas as pl
from jax.experimental.pallas import tpu as pltpu
from jax.experimental.pallas import tpu_sc as plsc

NUM_CORES = 2
NUM_SUBCORES = 16
NUM_TILES = NUM_CORES * NUM_SUBCORES
CHUNK = 1024
SUBCHUNKS = 2
STAGE = CHUNK * SUBCHUNKS
LANES = 16


def _make_sc_call(N: int, nnz_pad: int):
    per_tile = nnz_pad // NUM_TILES
    n_stage = per_tile // STAGE
    rows_per_tile = N // NUM_SUBCORES

    mesh = plsc.VectorSubcoreMesh(
        core_axis_name="c", subcore_axis_name="s", num_cores=NUM_CORES
    )

    @functools.partial(
        pl.kernel,
        out_type=jax.ShapeDtypeStruct((NUM_CORES, N, LANES), jnp.float32),
        mesh=mesh,
        scratch_types=dict(
            acc=pltpu.MemorySpace.VMEM_SHARED((N, LANES), jnp.float32),
            cidx=pltpu.MemorySpace.VMEM((SUBCHUNKS, CHUNK), jnp.int32),
            ridx=pltpu.MemorySpace.VMEM((SUBCHUNKS, CHUNK), jnp.int32),
            vals=pltpu.MemorySpace.VMEM((STAGE,), jnp.float32),
            gbuf=pltpu.MemorySpace.VMEM((CHUNK, LANES), jnp.float32),
            pbuf=pltpu.MemorySpace.VMEM((CHUNK, LANES), jnp.float32),
            sem=pltpu.SemaphoreType.DMA,
        ),
        compiler_params=pltpu.CompilerParams(use_tc_tiling_on_sc=False),
    )
    def sc_call(x16, rows_h, cols_h, vals_h, out_h, acc, cidx, ridx, vals,
                gbuf, pbuf, sem):
        c = lax.axis_index("c")
        s = lax.axis_index("s")

        z16 = jnp.zeros((LANES,), jnp.float32)

        for i in range(CHUNK):
            pbuf[i] = z16
        for i in range(rows_per_tile // CHUNK):
            pltpu.sync_copy(
                pbuf, acc.at[pl.ds(s * rows_per_tile + i * CHUNK, CHUNK)]
            )
        plsc.subcore_barrier()

        tile_id = s * NUM_CORES + c
        cbase = tile_id * (per_tile // CHUNK)

        def stage_body(t, _):
            coff = cbase + t * SUBCHUNKS
            pltpu.sync_copy(rows_h.at[pl.ds(coff, SUBCHUNKS)], ridx)
            pltpu.sync_copy(cols_h.at[pl.ds(coff, SUBCHUNKS)], cidx)
            pltpu.sync_copy(
                vals_h.at[pl.ds((cbase + t * SUBCHUNKS) * CHUNK, STAGE)], vals
            )

            def chunk_body(j, _):
                pltpu.async_copy(x16.at[cidx.at[j]], gbuf, sem).wait()
                for i16 in range(CHUNK // 16):
                    vv = vals[pl.ds(j * CHUNK + i16 * 16, 16)]
                    for u in range(16):
                        row = i16 * 16 + u
                        pbuf[row] = gbuf[row] * vv[u]
                pltpu.sync_copy(pbuf, acc.at[ridx.at[j]], add=True)
                return 0

            lax.fori_loop(0, SUBCHUNKS, chunk_body, 0)
            return 0

        lax.fori_loop(0, n_stage, stage_body, 0)

        plsc.subcore_barrier()
        def out_body(i, _):
            sl = pl.ds(s * rows_per_tile + i * CHUNK, CHUNK)
            pltpu.sync_copy(acc.at[sl], gbuf)
            pltpu.sync_copy(gbuf, out_h.at[c].at[sl])
            return 0

        lax.fori_loop(0, rows_per_tile // CHUNK, out_body, 0)

    return sc_call


def _combine(partials):
    two, n, lanes = partials.shape

    def body(p_ref, o_ref):
        o_ref[...] = p_ref[0] + p_ref[1]

    return pl.pallas_call(
        body,
        out_shape=jax.ShapeDtypeStruct((n, lanes), jnp.float32),
    )(partials)


def kernel(x_batched, rows, cols, values):
    B, N = x_batched.shape
    nnz = rows.shape[0]

    unit = NUM_TILES * STAGE
    nnz_pad = ((nnz + unit - 1) // unit) * unit
    pad = nnz_pad - nnz
    if pad:
        rows = jnp.concatenate([rows, jnp.zeros((pad,), rows.dtype)])
        cols = jnp.concatenate([cols, jnp.zeros((pad,), cols.dtype)])
        values = jnp.concatenate([values, jnp.zeros((pad,), values.dtype)])

    x16 = jnp.concatenate(
        [x_batched.T, jnp.zeros((N, LANES - B), jnp.float32)], axis=1
    )

    rows2d = rows.reshape(-1, CHUNK)
    cols2d = cols.reshape(-1, CHUNK)
    out2 = _make_sc_call(N, nnz_pad)(x16, rows2d, cols2d, values)
    out = _combine(out2)
    return out[:, :B].T


# speedup vs baseline: 1.4644x; 1.4644x over previous
import functools

import jax
import jax.numpy as jnp
from jax import lax
from jax.experimental import pallas as pl
from jax.experimental.pallas import tpu as pltpu
from jax.experimental.pallas import tpu_sc as plsc

NUM_CORES = 2
NUM_SUBCORES = 16
NUM_TILES = NUM_CORES * NUM_SUBCORES
CHUNK = 512
SUBCHUNKS = 4
STAGE = CHUNK * SUBCHUNKS
LANES = 16


def _make_sc_call(N: int, nnz_pad: int):
    per_tile = nnz_pad // NUM_TILES
    n_stage = per_tile // STAGE
    rows_per_tile = N // NUM_SUBCORES

    mesh = plsc.VectorSubcoreMesh(
        core_axis_name="c", subcore_axis_name="s", num_cores=NUM_CORES
    )

    @functools.partial(
        pl.kernel,
        out_type=jax.ShapeDtypeStruct((NUM_CORES, N, LANES), jnp.float32),
        mesh=mesh,
        scratch_types=dict(
            acc=pltpu.MemorySpace.VMEM_SHARED((N, LANES), jnp.float32),
            cidx=pltpu.MemorySpace.VMEM((SUBCHUNKS, CHUNK), jnp.int32),
            ridx=pltpu.MemorySpace.VMEM((SUBCHUNKS, CHUNK), jnp.int32),
            vals=pltpu.MemorySpace.VMEM((STAGE,), jnp.float32),
            g0=pltpu.MemorySpace.VMEM((CHUNK, LANES), jnp.float32),
            g1=pltpu.MemorySpace.VMEM((CHUNK, LANES), jnp.float32),
            lsem=pltpu.SemaphoreType.DMA,
            gsem0=pltpu.SemaphoreType.DMA,
            gsem1=pltpu.SemaphoreType.DMA,
            ssem0=pltpu.SemaphoreType.DMA,
            ssem1=pltpu.SemaphoreType.DMA,
        ),
        compiler_params=pltpu.CompilerParams(use_tc_tiling_on_sc=False),
    )
    def sc_call(x16, rows_h, cols_h, vals_h, out_h, acc, cidx, ridx, vals,
                g0, g1, lsem, gsem0, gsem1, ssem0, ssem1):
        c = lax.axis_index("c")
        s = lax.axis_index("s")

        z16 = jnp.zeros((LANES,), jnp.float32)

        for i in range(CHUNK):
            g0[i] = z16
        for i in range(rows_per_tile // CHUNK):
            pltpu.sync_copy(
                g0, acc.at[pl.ds(s * rows_per_tile + i * CHUNK, CHUNK)]
            )
        plsc.subcore_barrier()

        tile_id = s * NUM_CORES + c
        cbase = tile_id * (per_tile // CHUNK)

        gbufs = (g0, g1)
        gsems = (gsem0, gsem1)
        ssems = (ssem0, ssem1)

        def compute_inplace(gb, j):
            for i16 in range(CHUNK // 16):
                vv = vals[pl.ds(j * CHUNK + i16 * 16, 16)]
                for u in range(16):
                    row = i16 * 16 + u
                    gb[row] = gb[row] * vv[u]

        def stage_body(t, _):
            coff = cbase + t * SUBCHUNKS
            d1 = pltpu.async_copy(rows_h.at[pl.ds(coff, SUBCHUNKS)], ridx, lsem)
            d2 = pltpu.async_copy(cols_h.at[pl.ds(coff, SUBCHUNKS)], cidx, lsem)
            d3 = pltpu.async_copy(
                vals_h.at[pl.ds(coff * CHUNK, STAGE)], vals, lsem
            )
            d1.wait()
            d2.wait()
            d3.wait()

            gd = [None, None]
            sd = [None, None]
            gd[0] = pltpu.async_copy(x16.at[cidx.at[0]], g0, gsem0)
            for j in range(SUBCHUNKS):
                b = j & 1
                gd[b].wait()
                if j + 1 < SUBCHUNKS:
                    if sd[1 - b] is not None:
                        sd[1 - b].wait()
                    gd[1 - b] = pltpu.async_copy(
                        x16.at[cidx.at[j + 1]], gbufs[1 - b], gsems[1 - b]
                    )
                compute_inplace(gbufs[b], j)
                sd[b] = pltpu.async_copy(
                    gbufs[b], acc.at[ridx.at[j]], ssems[b], add=True
                )
            sd[0].wait()
            sd[1].wait()
            return 0

        lax.fori_loop(0, n_stage, stage_body, 0)

        plsc.subcore_barrier()
        def out_body(i, _):
            sl = pl.ds(s * rows_per_tile + i * CHUNK, CHUNK)
            pltpu.sync_copy(acc.at[sl], g0)
            pltpu.sync_copy(g0, out_h.at[c].at[sl])
            return 0

        lax.fori_loop(0, rows_per_tile // CHUNK, out_body, 0)

    return sc_call


def _combine(partials):
    two, n, lanes = partials.shape

    def body(p_ref, o_ref):
        o_ref[...] = p_ref[0] + p_ref[1]

    return pl.pallas_call(
        body,
        out_shape=jax.ShapeDtypeStruct((n, lanes), jnp.float32),
    )(partials)


def kernel(x_batched, rows, cols, values):
    B, N = x_batched.shape
    nnz = rows.shape[0]

    unit = NUM_TILES * STAGE
    nnz_pad = ((nnz + unit - 1) // unit) * unit
    pad = nnz_pad - nnz
    if pad:
        rows = jnp.concatenate([rows, jnp.zeros((pad,), rows.dtype)])
        cols = jnp.concatenate([cols, jnp.zeros((pad,), cols.dtype)])
        values = jnp.concatenate([values, jnp.zeros((pad,), values.dtype)])

    x16 = jnp.concatenate(
        [x_batched.T, jnp.zeros((N, LANES - B), jnp.float32)], axis=1
    )

    rows2d = rows.reshape(-1, CHUNK)
    cols2d = cols.reshape(-1, CHUNK)
    out2 = _make_sc_call(N, nnz_pad)(x16, rows2d, cols2d, values)
    out = _combine(out2)
    return out[:, :B].T
